# baseline (speedup 1.0000x reference)
import functools

import jax
import jax.numpy as jnp
from jax import lax
from jax.experimental import pallas as pl
from jax.experimental.pallas import tpu as pltpu
from jax.experimental.pallas import tpu_sc as plsc

N_NODES = 10000
D = 128
NS = 16
NC = 2
ACC_ROWS = 10240
CHUNK = 128
ZROWS = ACC_ROWS // NS
PAD_DST = N_NODES + 64
N0_CHUNKS = 48
N1_CHUNKS = 112


def _make_agg():
    mesh = plsc.VectorSubcoreMesh(core_axis_name="c", subcore_axis_name="s")

    @functools.partial(
        pl.kernel,
        mesh=mesh,
        out_type=jax.ShapeDtypeStruct((NC, N_NODES, D), jnp.float32),
        scratch_types=[
            pltpu.VMEM((CHUNK,), jnp.int32),
            pltpu.VMEM((CHUNK,), jnp.int32),
            pltpu.VMEM((CHUNK, D), jnp.float32),
            pltpu.VMEM_SHARED((ACC_ROWS, D), jnp.float32),
            pltpu.SemaphoreType.DMA,
        ],
    )
    def agg(x_hbm, src_hbm, dst_hbm, zero_hbm, out_hbm,
            src_v, dst_v, rows_v, acc, sem):
        c = lax.axis_index("c")
        s = lax.axis_index("s")

        pltpu.sync_copy(zero_hbm, acc.at[pl.ds(s * ZROWS, ZROWS)])
        plsc.subcore_barrier()

        n_chunks = jnp.where(c == 0, N0_CHUNKS, N1_CHUNKS)
        base = jnp.where(c == 0, s * (N0_CHUNKS * CHUNK),
                         NS * (N0_CHUNKS * CHUNK) + s * (N1_CHUNKS * CHUNK))

        def body(k, carry):
            off = base + k * CHUNK
            pltpu.sync_copy(src_hbm.at[pl.ds(off, CHUNK)], src_v)
            pltpu.sync_copy(dst_hbm.at[pl.ds(off, CHUNK)], dst_v)
            pltpu.async_copy(x_hbm.at[src_v], rows_v, sem).wait()
            pltpu.sync_copy(rows_v, acc.at[dst_v], add=True)
            return carry

        lax.fori_loop(0, n_chunks, body, 0)
        plsc.subcore_barrier()

        r0 = s * ZROWS

        @pl.when(s < NS - 1)
        def _():
            pltpu.sync_copy(acc.at[pl.ds(r0, ZROWS)],
                            out_hbm.at[c, pl.ds(r0, ZROWS)])

        @pl.when(s == NS - 1)
        def _():
            pltpu.sync_copy(acc.at[pl.ds(r0, N_NODES - (NS - 1) * ZROWS)],
                            out_hbm.at[c, pl.ds(r0, N_NODES - (NS - 1) * ZROWS)])

    return agg


def _mlp_body(x_ref, p_ref, w1_ref, b1_ref, g_ref, be_ref, w2_ref, b2_ref,
              o_ref):
    h = x_ref[...] + p_ref[0] + p_ref[1]
    h1 = lax.dot_general(h, w1_ref[...], (((1,), (1,)), ((), ())),
                         preferred_element_type=jnp.float32) + b1_ref[...]
    mean = jnp.mean(h1, axis=0, keepdims=True)
    d = h1 - mean
    var = jnp.mean(d * d, axis=0, keepdims=True)
    hn = d * (lax.rsqrt(var + 1e-5) * g_ref[...]) + be_ref[...]
    hr = jnp.maximum(hn, 0.0)
    o_ref[...] = lax.dot_general(hr, w2_ref[...], (((1,), (1,)), ((), ())),
                                 preferred_element_type=jnp.float32) + b2_ref[...]


def kernel(x, edge_index, edge_attr, W1, b1, gamma, beta, W2, b2):
    del edge_attr
    src = edge_index[0].astype(jnp.int32)
    dst = edge_index[1].astype(jnp.int32)
    e = src.shape[0]
    e_pad = NS * (N0_CHUNKS + N1_CHUNKS) * CHUNK
    assert e_pad >= e
    if e_pad != e:
        pad = e_pad - e
        src = jnp.concatenate([src, jnp.zeros((pad,), jnp.int32)])
        dst = jnp.concatenate([dst, jnp.full((pad,), PAD_DST, jnp.int32)])
    zeros = jnp.zeros((ZROWS, D), jnp.float32)

    parts = _make_agg()(x, src, dst, zeros)

    return pl.pallas_call(
        _mlp_body,
        out_shape=jax.ShapeDtypeStruct((N_NODES, D), jnp.float32),
    )(x, parts, W1, b1.reshape(1, D), gamma.reshape(1, D), beta.reshape(1, D),
      W2, b2.reshape(1, D))


# speedup vs baseline: 1.5321x; 1.5321x over previous
import functools

import jax
import jax.numpy as jnp
from jax import lax
from jax.experimental import pallas as pl
from jax.experimental.pallas import tpu as pltpu
from jax.experimental.pallas import tpu_sc as plsc

N_NODES = 10000
D = 128
NS = 16
NC = 2
ACC_ROWS = 10240
CHUNK = 128
ZROWS = ACC_ROWS // NS
PAD_DST = N_NODES + 64
N0_CHUNKS = 128
N1_CHUNKS = 32


def _make_agg():
    mesh = plsc.VectorSubcoreMesh(core_axis_name="c", subcore_axis_name="s")

    @functools.partial(
        pl.kernel,
        mesh=mesh,
        out_type=jax.ShapeDtypeStruct((NC, N_NODES, D), jnp.float32),
        scratch_types=[
            pltpu.VMEM((CHUNK,), jnp.int32),
            pltpu.VMEM((CHUNK,), jnp.int32),
            pltpu.VMEM((CHUNK, D), jnp.float32),
            pltpu.VMEM_SHARED((ACC_ROWS, D), jnp.float32),
            pltpu.SemaphoreType.DMA,
        ],
    )
    def agg(x_hbm, src_hbm, dst_hbm, zero_hbm, out_hbm,
            src_v, dst_v, rows_v, acc, sem):
        c = lax.axis_index("c")
        s = lax.axis_index("s")

        pltpu.sync_copy(zero_hbm, acc.at[pl.ds(s * ZROWS, ZROWS)])
        plsc.subcore_barrier()

        n_chunks = jnp.where(c == 0, N0_CHUNKS, N1_CHUNKS)
        base = jnp.where(c == 0, s * (N0_CHUNKS * CHUNK),
                         NS * (N0_CHUNKS * CHUNK) + s * (N1_CHUNKS * CHUNK))

        def body(k, carry):
            off = base + k * CHUNK
            pltpu.sync_copy(src_hbm.at[pl.ds(off, CHUNK)], src_v)
            pltpu.sync_copy(dst_hbm.at[pl.ds(off, CHUNK)], dst_v)
            pltpu.async_copy(x_hbm.at[src_v], rows_v, sem).wait()
            pltpu.sync_copy(rows_v, acc.at[dst_v], add=True)
            return carry

        lax.fori_loop(0, n_chunks, body, 0)
        plsc.subcore_barrier()

        r0 = s * ZROWS

        @pl.when(s < NS - 1)
        def _():
            pltpu.sync_copy(acc.at[pl.ds(r0, ZROWS)],
                            out_hbm.at[c, pl.ds(r0, ZROWS)])

        @pl.when(s == NS - 1)
        def _():
            pltpu.sync_copy(acc.at[pl.ds(r0, N_NODES - (NS - 1) * ZROWS)],
                            out_hbm.at[c, pl.ds(r0, N_NODES - (NS - 1) * ZROWS)])

    return agg


def _mlp_body(x_ref, p_ref, w1_ref, b1_ref, g_ref, be_ref, w2_ref, b2_ref,
              o_ref):
    h = x_ref[...] + p_ref[0] + p_ref[1]
    h1 = lax.dot_general(h, w1_ref[...], (((1,), (1,)), ((), ())),
                         preferred_element_type=jnp.float32) + b1_ref[...]
    mean = jnp.mean(h1, axis=0, keepdims=True)
    d = h1 - mean
    var = jnp.mean(d * d, axis=0, keepdims=True)
    hn = d * (lax.rsqrt(var + 1e-5) * g_ref[...]) + be_ref[...]
    hr = jnp.maximum(hn, 0.0)
    o_ref[...] = lax.dot_general(hr, w2_ref[...], (((1,), (1,)), ((), ())),
                                 preferred_element_type=jnp.float32) + b2_ref[...]


def kernel(x, edge_index, edge_attr, W1, b1, gamma, beta, W2, b2):
    del edge_attr
    src = edge_index[0].astype(jnp.int32)
    dst = edge_index[1].astype(jnp.int32)
    e = src.shape[0]
    e_pad = NS * (N0_CHUNKS + N1_CHUNKS) * CHUNK
    assert e_pad >= e
    if e_pad != e:
        pad = e_pad - e
        src = jnp.concatenate([src, jnp.zeros((pad,), jnp.int32)])
        dst = jnp.concatenate([dst, jnp.full((pad,), PAD_DST, jnp.int32)])
    zeros = jnp.zeros((ZROWS, D), jnp.float32)

    parts = _make_agg()(x, src, dst, zeros)

    return pl.pallas_call(
        _mlp_body,
        out_shape=jax.ShapeDtypeStruct((N_NODES, D), jnp.float32),
    )(x, parts, W1, b1.reshape(1, D), gamma.reshape(1, D), beta.reshape(1, D),
      W2, b2.reshape(1, D))


# speedup vs baseline: 1.6386x; 1.0695x over previous
import functools

import jax
import jax.numpy as jnp
from jax import lax
from jax.experimental import pallas as pl
from jax.experimental.pallas import tpu as pltpu
from jax.experimental.pallas import tpu_sc as plsc

N_NODES = 10000
D = 128
NS = 16
NC = 2
ACC_ROWS = 10240
CHUNK = 128
ROWS_PT = ACC_ROWS // NS
TAIL_ROWS = N_NODES - (NS - 1) * ROWS_PT
PAD_DST = N_NODES + 64


def _make_agg(n_chunks):
    mesh = plsc.VectorSubcoreMesh(core_axis_name="c", subcore_axis_name="s")
    ept = n_chunks * CHUNK

    @functools.partial(
        pl.kernel,
        mesh=mesh,
        out_type=jax.ShapeDtypeStruct((NC, N_NODES, D), jnp.float32),
        scratch_types=[
            pltpu.VMEM((CHUNK,), jnp.int32),
            pltpu.VMEM((CHUNK,), jnp.int32),
            pltpu.VMEM((CHUNK, D), jnp.float32),
            pltpu.VMEM_SHARED((ACC_ROWS, D), jnp.float32),
            pltpu.SemaphoreType.DMA,
        ],
    )
    def agg(x_hbm, src_hbm, dst_hbm, zero_hbm, out_hbm,
            src_v, dst_v, rows_v, acc, sem):
        c = lax.axis_index("c")
        s = lax.axis_index("s")
        wid = c * NS + s
        r0 = s * ROWS_PT

        @pl.when(jnp.logical_and(c == 0, s < NS - 1))
        def _():
            pltpu.sync_copy(x_hbm.at[pl.ds(r0, ROWS_PT)],
                            acc.at[pl.ds(r0, ROWS_PT)])

        @pl.when(jnp.logical_and(c == 0, s == NS - 1))
        def _():
            pltpu.sync_copy(x_hbm.at[pl.ds(r0, TAIL_ROWS)],
                            acc.at[pl.ds(r0, TAIL_ROWS)])

        @pl.when(c != 0)
        def _():
            pltpu.sync_copy(zero_hbm, acc.at[pl.ds(r0, ROWS_PT)])

        plsc.subcore_barrier()

        base = wid * ept

        def body(k, carry):
            off = base + k * CHUNK
            pltpu.sync_copy(src_hbm.at[pl.ds(off, CHUNK)], src_v)
            pltpu.sync_copy(dst_hbm.at[pl.ds(off, CHUNK)], dst_v)
            pltpu.async_copy(x_hbm.at[src_v], rows_v, sem).wait()
            pltpu.sync_copy(rows_v, acc.at[dst_v], add=True)
            return carry

        lax.fori_loop(0, n_chunks, body, 0)
        plsc.subcore_barrier()

        @pl.when(s < NS - 1)
        def _():
            pltpu.sync_copy(acc.at[pl.ds(r0, ROWS_PT)],
                            out_hbm.at[c, pl.ds(r0, ROWS_PT)])

        @pl.when(s == NS - 1)
        def _():
            pltpu.sync_copy(acc.at[pl.ds(r0, TAIL_ROWS)],
                            out_hbm.at[c, pl.ds(r0, TAIL_ROWS)])

    return agg


def _mlp_body(p_ref, w1_ref, b1_ref, g_ref, be_ref, w2_ref, b2_ref, o_ref):
    h = p_ref[0] + p_ref[1]
    h1 = lax.dot_general(h, w1_ref[...], (((1,), (1,)), ((), ())),
                         preferred_element_type=jnp.float32) + b1_ref[...]
    mean = jnp.mean(h1, axis=0, keepdims=True)
    d = h1 - mean
    var = jnp.mean(d * d, axis=0, keepdims=True)
    hn = d * (lax.rsqrt(var + 1e-5) * g_ref[...]) + be_ref[...]
    hr = jnp.maximum(hn, 0.0)
    o_ref[...] = lax.dot_general(hr, w2_ref[...], (((1,), (1,)), ((), ())),
                                 preferred_element_type=jnp.float32) + b2_ref[...]


def kernel(x, edge_index, edge_attr, W1, b1, gamma, beta, W2, b2):
    del edge_attr
    src = edge_index[0].astype(jnp.int32)
    dst = edge_index[1].astype(jnp.int32)
    e = src.shape[0]
    grain = 32 * CHUNK
    e_pad = ((e + grain - 1) // grain) * grain
    if e_pad != e:
        pad = e_pad - e
        src = jnp.concatenate([src, jnp.zeros((pad,), jnp.int32)])
        dst = jnp.concatenate([dst, jnp.full((pad,), PAD_DST, jnp.int32)])
    n_chunks = e_pad // grain
    zeros = jnp.zeros((ACC_ROWS // NS, D), jnp.float32)

    parts = _make_agg(n_chunks)(x, src, dst, zeros)

    return pl.pallas_call(
        _mlp_body,
        out_shape=jax.ShapeDtypeStruct((N_NODES, D), jnp.float32),
    )(parts, W1, b1.reshape(1, D), gamma.reshape(1, D), beta.reshape(1, D),
      W2, b2.reshape(1, D))
